# Initial kernel scaffold; baseline (speedup 1.0000x reference)
import jax
import jax.numpy as jnp
from jax.experimental import pallas as pl


def kernel(u, centroids, Wr, br, Ws, bs):
    raise NotImplementedError("write your pallas kernel here")


# speedup vs baseline: 1.7565x; 1.7565x over previous
import jax
import jax.numpy as jnp
from jax.experimental import pallas as pl
from jax.experimental.pallas import tpu as pltpu

B, T, D = 2, 2048, 1024
NR, NS, TOPK = 16, 2, 2
M = B * T
TM = 512


def _moe_body(cent_ref, wr_ref, br_ref, ws_ref, bsum_ref,
              u_ref, ubf_ref, o_ref):
    u = u_ref[...]
    ubf = ubf_ref[...]

    s = jax.nn.sigmoid(
        jnp.dot(u, cent_ref[...], preferred_element_type=jnp.float32))

    idx = jax.lax.broadcasted_iota(jnp.int32, s.shape, 1)
    m1 = jnp.max(s, axis=1, keepdims=True)
    i1 = jnp.min(jnp.where(s == m1, idx, NR), axis=1, keepdims=True)
    s2 = jnp.where(idx == i1, -jnp.inf, s)
    m2 = jnp.max(s2, axis=1, keepdims=True)
    i2 = jnp.min(jnp.where(s2 == m2, idx, NR), axis=1, keepdims=True)
    gmat = jnp.where((idx == i1) | (idx == i2), s, 0.0)

    acc = u + bsum_ref[...] + jnp.dot(gmat, br_ref[...],
                                      preferred_element_type=jnp.float32)
    acc = acc + jnp.dot(ubf, ws_ref[...], preferred_element_type=jnp.float32)
    for e in range(NR):
        pe = jnp.dot(ubf, wr_ref[e], preferred_element_type=jnp.float32)
        acc = acc + gmat[:, e:e + 1] * pe
    o_ref[...] = acc


def kernel(u, centroids, Wr, br, Ws, bs):
    uf = u.reshape(M, D)
    ubf = uf.astype(jnp.bfloat16)
    centT = centroids.T
    wrT = jnp.transpose(Wr, (0, 2, 1)).astype(jnp.bfloat16)
    wsT = (Ws[0] + Ws[1]).T.astype(jnp.bfloat16)
    bsum = (bs[0] + bs[1]).reshape(1, D)

    grid = (M // TM,)
    out = pl.pallas_call(
        _moe_body,
        grid=grid,
        in_specs=[
            pl.BlockSpec((D, NR), lambda i: (0, 0)),
            pl.BlockSpec((NR, D, D), lambda i: (0, 0, 0)),
            pl.BlockSpec((NR, D), lambda i: (0, 0)),
            pl.BlockSpec((D, D), lambda i: (0, 0)),
            pl.BlockSpec((1, D), lambda i: (0, 0)),
            pl.BlockSpec((TM, D), lambda i: (i, 0)),
            pl.BlockSpec((TM, D), lambda i: (i, 0)),
        ],
        out_specs=pl.BlockSpec((TM, D), lambda i: (i, 0)),
        out_shape=jax.ShapeDtypeStruct((M, D), jnp.float32),
    )(centT, wrT, br, wsT, bsum, uf, ubf)
    return out.reshape(B, T, D)


# speedup vs baseline: 1.9107x; 1.0878x over previous
import jax
import jax.numpy as jnp
from jax.experimental import pallas as pl
from jax.experimental.pallas import tpu as pltpu

B, T, D = 2, 2048, 1024
NR, NS, TOPK = 16, 2, 2
M = B * T
TM = 512


def _moe_body(cent_ref, wr_ref, br_ref, ws_ref, bsum_ref,
              u_ref, ubf_ref, o_ref):
    u = u_ref[...]
    ubf = ubf_ref[...]

    s = jax.nn.sigmoid(
        jnp.dot(u, cent_ref[...], preferred_element_type=jnp.float32))

    idx = jax.lax.broadcasted_iota(jnp.int32, s.shape, 1)
    m1 = jnp.max(s, axis=1, keepdims=True)
    i1 = jnp.min(jnp.where(s == m1, idx, NR), axis=1, keepdims=True)
    s2 = jnp.where(idx == i1, -jnp.inf, s)
    m2 = jnp.max(s2, axis=1, keepdims=True)
    i2 = jnp.min(jnp.where(s2 == m2, idx, NR), axis=1, keepdims=True)
    gmat = jnp.where((idx == i1) | (idx == i2), s, 0.0)

    acc = u + bsum_ref[...] + jnp.dot(gmat, br_ref[...],
                                      preferred_element_type=jnp.float32)
    acc = acc + jnp.dot(ubf, ws_ref[...], preferred_element_type=jnp.float32)
    for e in range(NR):
        pe = jax.lax.dot_general(
            ubf, wr_ref[e], (((1,), (1,)), ((), ())),
            preferred_element_type=jnp.float32)
        acc = acc + gmat[:, e:e + 1] * pe
    o_ref[...] = acc


def kernel(u, centroids, Wr, br, Ws, bs):
    uf = u.reshape(M, D)
    ubf = uf.astype(jnp.bfloat16)
    centT = centroids.T
    wrT = Wr.astype(jnp.bfloat16)
    wsT = (Ws[0] + Ws[1]).T.astype(jnp.bfloat16)
    bsum = (bs[0] + bs[1]).reshape(1, D)

    grid = (M // TM,)
    out = pl.pallas_call(
        _moe_body,
        grid=grid,
        in_specs=[
            pl.BlockSpec((D, NR), lambda i: (0, 0)),
            pl.BlockSpec((NR, D, D), lambda i: (0, 0, 0)),
            pl.BlockSpec((NR, D), lambda i: (0, 0)),
            pl.BlockSpec((D, D), lambda i: (0, 0)),
            pl.BlockSpec((1, D), lambda i: (0, 0)),
            pl.BlockSpec((TM, D), lambda i: (i, 0)),
            pl.BlockSpec((TM, D), lambda i: (i, 0)),
        ],
        out_specs=pl.BlockSpec((TM, D), lambda i: (i, 0)),
        out_shape=jax.ShapeDtypeStruct((M, D), jnp.float32),
    )(centT, wrT, br, wsT, bsum, uf, ubf)
    return out.reshape(B, T, D)
